# Initial kernel scaffold; baseline (speedup 1.0000x reference)
import jax
import jax.numpy as jnp
from jax.experimental import pallas as pl


def kernel(node_features, edge_index):
    raise NotImplementedError("write your pallas kernel here")


# speedup vs baseline: 7.0500x; 7.0500x over previous
import functools

import jax
import jax.numpy as jnp
from jax import lax
from jax.experimental import pallas as pl
from jax.experimental.pallas import tpu as pltpu
from jax.experimental.pallas import tpu_sc as plsc

N_NODES = 10000
N_EDGES = 320000
D_FEAT = 128

NC = 2
NS = 16
NW = NC * NS
CHUNK = 80
EDGES_PER_W = N_EDGES // NW
CHUNKS_PER_W = EDGES_PER_W // CHUNK
N_PAD = 10240
ROWS_PER_TILE = N_PAD // NS

_mesh = plsc.VectorSubcoreMesh(core_axis_name="c", subcore_axis_name="s")


@functools.partial(
    pl.kernel,
    mesh=_mesh,
    compiler_params=pltpu.CompilerParams(use_tc_tiling_on_sc=False),
    out_type=jax.ShapeDtypeStruct((NC, N_PAD, D_FEAT), jnp.float32),
    scratch_types=[
        pltpu.VMEM((CHUNKS_PER_W, CHUNK), jnp.int32),
        pltpu.VMEM((CHUNKS_PER_W, CHUNK), jnp.int32),
        pltpu.VMEM((CHUNK, D_FEAT), jnp.float32),
        pltpu.VMEM_SHARED((N_PAD, D_FEAT), jnp.float32),
        pltpu.SemaphoreType.DMA,
    ],
)
def _sc_aggregate(nodes_hbm, src_hbm, dst_hbm, zeros_hbm, out_hbm,
                  src_v, dst_v, rows_v, acc, sem):
    c = lax.axis_index("c")
    s = lax.axis_index("s")
    wid = c * NS + s
    row0 = s * ROWS_PER_TILE

    pltpu.sync_copy(zeros_hbm.at[pl.ds(row0, ROWS_PER_TILE)],
                    acc.at[pl.ds(row0, ROWS_PER_TILE)])
    chunk0 = wid * CHUNKS_PER_W
    pltpu.sync_copy(src_hbm.at[pl.ds(chunk0, CHUNKS_PER_W)], src_v)
    pltpu.sync_copy(dst_hbm.at[pl.ds(chunk0, CHUNKS_PER_W)], dst_v)
    plsc.subcore_barrier()

    def body(j, carry):
        pltpu.async_copy(nodes_hbm.at[src_v.at[j]], rows_v, sem).wait()
        pltpu.sync_copy(rows_v, acc.at[dst_v.at[j]], add=True)
        return carry

    lax.fori_loop(0, CHUNKS_PER_W, body, 0)

    plsc.subcore_barrier()
    pltpu.sync_copy(acc.at[pl.ds(row0, ROWS_PER_TILE)],
                    out_hbm.at[c, pl.ds(row0, ROWS_PER_TILE)])


def _combine_body(p_ref, o_ref):
    o_ref[...] = p_ref[0] + p_ref[1]


_ROWS_BLK = 1000


def _combine(partials):
    return pl.pallas_call(
        _combine_body,
        grid=(N_NODES // _ROWS_BLK,),
        in_specs=[pl.BlockSpec((NC, _ROWS_BLK, D_FEAT), lambda i: (0, i, 0))],
        out_specs=pl.BlockSpec((_ROWS_BLK, D_FEAT), lambda i: (i, 0)),
        out_shape=jax.ShapeDtypeStruct((N_NODES, D_FEAT), jnp.float32),
    )(partials)


def kernel(node_features, edge_index):
    src = edge_index[0].astype(jnp.int32).reshape(N_EDGES // CHUNK, CHUNK)
    dst = edge_index[1].astype(jnp.int32).reshape(N_EDGES // CHUNK, CHUNK)
    zeros = jnp.zeros((N_PAD, D_FEAT), jnp.float32)
    partials = _sc_aggregate(node_features, src, dst, zeros)
    return _combine(partials)


# speedup vs baseline: 10.8345x; 1.5368x over previous
import functools

import jax
import jax.numpy as jnp
from jax import lax
from jax.experimental import pallas as pl
from jax.experimental.pallas import tpu as pltpu
from jax.experimental.pallas import tpu_sc as plsc

N_NODES = 10000
N_EDGES = 320000
D_FEAT = 128

NC = 2
NS = 16
NW = NC * NS
CHUNK = 80
EDGES_PER_W = N_EDGES // NW
CHUNKS_PER_W = EDGES_PER_W // CHUNK
N_PAD = 10240
ROWS_PER_TILE = N_PAD // NS

_mesh = plsc.VectorSubcoreMesh(core_axis_name="c", subcore_axis_name="s")


@functools.partial(
    pl.kernel,
    mesh=_mesh,
    compiler_params=pltpu.CompilerParams(use_tc_tiling_on_sc=False),
    out_type=jax.ShapeDtypeStruct((NC, N_PAD, D_FEAT), jnp.float32),
    scratch_types=[
        pltpu.VMEM((CHUNKS_PER_W, CHUNK), jnp.int32),
        pltpu.VMEM((CHUNKS_PER_W, CHUNK), jnp.int32),
        pltpu.VMEM((CHUNK, D_FEAT), jnp.float32),
        pltpu.VMEM((CHUNK, D_FEAT), jnp.float32),
        pltpu.VMEM_SHARED((N_PAD, D_FEAT), jnp.float32),
        pltpu.SemaphoreType.DMA,
        pltpu.SemaphoreType.DMA,
    ],
)
def _sc_aggregate(nodes_hbm, src_hbm, dst_hbm, zeros_hbm, out_hbm,
                  src_v, dst_v, rows0, rows1, acc, sem0, sem1):
    c = lax.axis_index("c")
    s = lax.axis_index("s")
    wid = c * NS + s
    row0 = s * ROWS_PER_TILE

    pltpu.sync_copy(zeros_hbm.at[pl.ds(row0, ROWS_PER_TILE)],
                    acc.at[pl.ds(row0, ROWS_PER_TILE)])
    chunk0 = wid * CHUNKS_PER_W
    pltpu.sync_copy(src_hbm.at[pl.ds(chunk0, CHUNKS_PER_W)], src_v)
    pltpu.sync_copy(dst_hbm.at[pl.ds(chunk0, CHUNKS_PER_W)], dst_v)
    plsc.subcore_barrier()

    bufs = (rows0, rows1)
    sems = (sem0, sem1)
    NB = 2

    for b in range(NB):
        pltpu.async_copy(nodes_hbm.at[src_v.at[b]], bufs[b], sems[b])

    @pl.loop(0, CHUNKS_PER_W - 1, step=NB)
    def _(g):
        for b in range(NB):
            j = g + b
            pltpu.make_async_copy(nodes_hbm.at[src_v.at[j]],
                                  bufs[b], sems[b]).wait()
            pltpu.sync_copy(bufs[b], acc.at[dst_v.at[j]], add=True)

            @pl.when(j + NB < CHUNKS_PER_W)
            def _issue():
                pltpu.async_copy(nodes_hbm.at[src_v.at[j + NB]],
                                 bufs[b], sems[b])

    tail = CHUNKS_PER_W - 1
    pltpu.make_async_copy(nodes_hbm.at[src_v.at[tail]], bufs[0], sems[0]).wait()
    pltpu.sync_copy(bufs[0], acc.at[dst_v.at[tail]], add=True)

    plsc.subcore_barrier()
    pltpu.sync_copy(acc.at[pl.ds(row0, ROWS_PER_TILE)],
                    out_hbm.at[c, pl.ds(row0, ROWS_PER_TILE)])


def _combine_body(p_ref, o_ref):
    o_ref[...] = p_ref[0] + p_ref[1]


_ROWS_BLK = 1000


def _combine(partials):
    return pl.pallas_call(
        _combine_body,
        grid=(N_NODES // _ROWS_BLK,),
        in_specs=[pl.BlockSpec((NC, _ROWS_BLK, D_FEAT), lambda i: (0, i, 0))],
        out_specs=pl.BlockSpec((_ROWS_BLK, D_FEAT), lambda i: (i, 0)),
        out_shape=jax.ShapeDtypeStruct((N_NODES, D_FEAT), jnp.float32),
    )(partials)


def kernel(node_features, edge_index):
    src = edge_index[0].astype(jnp.int32).reshape(N_EDGES // CHUNK, CHUNK)
    dst = edge_index[1].astype(jnp.int32).reshape(N_EDGES // CHUNK, CHUNK)
    zeros = jnp.zeros((N_PAD, D_FEAT), jnp.float32)
    partials = _sc_aggregate(node_features, src, dst, zeros)
    return _combine(partials)


# speedup vs baseline: 11.6722x; 1.0773x over previous
import functools

import jax
import jax.numpy as jnp
from jax import lax
from jax.experimental import pallas as pl
from jax.experimental.pallas import tpu as pltpu
from jax.experimental.pallas import tpu_sc as plsc

N_NODES = 10000
N_EDGES = 320000
D_FEAT = 128

NC = 2
NS = 16
NW = NC * NS
CHUNK = 40
EDGES_PER_W = N_EDGES // NW
CHUNKS_PER_W = EDGES_PER_W // CHUNK
N_PAD = 10240
ROWS_PER_TILE = N_PAD // NS
NB = 5

_mesh = plsc.VectorSubcoreMesh(core_axis_name="c", subcore_axis_name="s")


@functools.partial(
    pl.kernel,
    mesh=_mesh,
    compiler_params=pltpu.CompilerParams(use_tc_tiling_on_sc=False),
    out_type=jax.ShapeDtypeStruct((NC, N_PAD, D_FEAT), jnp.float32),
    scratch_types=[
        pltpu.VMEM((CHUNKS_PER_W, CHUNK), jnp.int32),
        pltpu.VMEM((CHUNKS_PER_W, CHUNK), jnp.int32),
        [pltpu.VMEM((CHUNK, D_FEAT), jnp.float32) for _ in range(NB)],
        pltpu.VMEM_SHARED((N_PAD, D_FEAT), jnp.float32),
        [pltpu.SemaphoreType.DMA for _ in range(NB)],
        [pltpu.SemaphoreType.DMA for _ in range(NB)],
    ],
)
def _sc_aggregate(nodes_hbm, src_hbm, dst_hbm, zeros_hbm, out_hbm,
                  src_v, dst_v, bufs, acc, gsems, ssems):
    c = lax.axis_index("c")
    s = lax.axis_index("s")
    wid = c * NS + s
    row0 = s * ROWS_PER_TILE

    pltpu.sync_copy(zeros_hbm.at[pl.ds(row0, ROWS_PER_TILE)],
                    acc.at[pl.ds(row0, ROWS_PER_TILE)])
    chunk0 = wid * CHUNKS_PER_W
    pltpu.sync_copy(src_hbm.at[pl.ds(chunk0, CHUNKS_PER_W)], src_v)
    pltpu.sync_copy(dst_hbm.at[pl.ds(chunk0, CHUNKS_PER_W)], dst_v)
    plsc.subcore_barrier()

    def gather(j, b):
        pltpu.async_copy(nodes_hbm.at[src_v.at[j]], bufs[b], gsems[b])

    def wait_gather(j, b):
        pltpu.make_async_copy(nodes_hbm.at[src_v.at[j]],
                              bufs[b], gsems[b]).wait()

    def scatter(j, b):
        pltpu.async_copy(bufs[b], acc.at[dst_v.at[j]], ssems[b], add=True)

    def wait_scatter(j, b):
        pltpu.make_async_copy(bufs[b], acc.at[dst_v.at[j]], ssems[b]).wait()

    for b in range(3):
        gather(b, b)

    @pl.loop(0, CHUNKS_PER_W, step=NB)
    def _(g):
        for b in range(NB):
            j = g + b
            b2 = (b + 3) % NB
            wait_gather(j, b)
            scatter(j, b)

            @pl.when(j < 2)
            def _fill():
                gather(j + 3, b2)

            @pl.when((j >= 2) & (j + 3 < CHUNKS_PER_W))
            def _recycle():
                wait_scatter(j - 2, b2)
                gather(j + 3, b2)

            @pl.when((j >= 2) & (j + 3 >= CHUNKS_PER_W))
            def _drain():
                wait_scatter(j - 2, b2)

    wait_scatter(CHUNKS_PER_W - 2, (CHUNKS_PER_W - 2) % NB)
    wait_scatter(CHUNKS_PER_W - 1, (CHUNKS_PER_W - 1) % NB)

    plsc.subcore_barrier()
    pltpu.sync_copy(acc.at[pl.ds(row0, ROWS_PER_TILE)],
                    out_hbm.at[c, pl.ds(row0, ROWS_PER_TILE)])


def _combine_body(p_ref, o_ref):
    o_ref[...] = p_ref[0] + p_ref[1]


_ROWS_BLK = 1000


def _combine(partials):
    return pl.pallas_call(
        _combine_body,
        grid=(N_NODES // _ROWS_BLK,),
        in_specs=[pl.BlockSpec((NC, _ROWS_BLK, D_FEAT), lambda i: (0, i, 0))],
        out_specs=pl.BlockSpec((_ROWS_BLK, D_FEAT), lambda i: (i, 0)),
        out_shape=jax.ShapeDtypeStruct((N_NODES, D_FEAT), jnp.float32),
    )(partials)


def kernel(node_features, edge_index):
    src = edge_index[0].astype(jnp.int32).reshape(N_EDGES // CHUNK, CHUNK)
    dst = edge_index[1].astype(jnp.int32).reshape(N_EDGES // CHUNK, CHUNK)
    zeros = jnp.zeros((N_PAD, D_FEAT), jnp.float32)
    partials = _sc_aggregate(node_features, src, dst, zeros)
    return _combine(partials)


# speedup vs baseline: 12.9618x; 1.1105x over previous
import functools

import jax
import jax.numpy as jnp
from jax import lax
from jax.experimental import pallas as pl
from jax.experimental.pallas import tpu as pltpu
from jax.experimental.pallas import tpu_sc as plsc

N_NODES = 10000
N_EDGES = 320000
D_FEAT = 128

NC = 2
NS = 16
NW = NC * NS
CHUNK = 40
EDGES_PER_W = N_EDGES // NW
CHUNKS_PER_W = EDGES_PER_W // CHUNK
N_PAD = 10240
ROWS_PER_TILE = N_PAD // NS
NB = 5

_mesh = plsc.VectorSubcoreMesh(core_axis_name="c", subcore_axis_name="s")


@functools.partial(
    pl.kernel,
    mesh=_mesh,
    compiler_params=pltpu.CompilerParams(use_tc_tiling_on_sc=False),
    out_type=jax.ShapeDtypeStruct((NC, N_PAD, D_FEAT), jnp.float32),
    scratch_types=[
        pltpu.VMEM((CHUNKS_PER_W, CHUNK), jnp.int32),
        pltpu.VMEM((CHUNKS_PER_W, CHUNK), jnp.int32),
        [pltpu.VMEM((CHUNK, D_FEAT), jnp.float32) for _ in range(NB)],
        pltpu.VMEM_SHARED((N_PAD, D_FEAT), jnp.float32),
        [pltpu.SemaphoreType.DMA for _ in range(NB)],
        [pltpu.SemaphoreType.DMA for _ in range(NB)],
    ],
)
def _sc_aggregate(nodes_hbm, edges_hbm, out_hbm,
                  src_v, dst_v, bufs, acc, gsems, ssems):
    c = lax.axis_index("c")
    s = lax.axis_index("s")
    wid = c * NS + s
    row0 = s * ROWS_PER_TILE

    @pl.loop(0, CHUNK)
    def _zrow(r):
        for c16 in range(D_FEAT // 16):
            bufs[0][r, pl.ds(c16 * 16, 16)] = jnp.zeros((16,), jnp.float32)
    for k in range(ROWS_PER_TILE // CHUNK):
        pltpu.sync_copy(bufs[0], acc.at[pl.ds(row0 + k * CHUNK, CHUNK)])
    chunk0 = wid * CHUNKS_PER_W
    pltpu.sync_copy(edges_hbm.at[0, pl.ds(chunk0, CHUNKS_PER_W)], src_v)
    pltpu.sync_copy(edges_hbm.at[1, pl.ds(chunk0, CHUNKS_PER_W)], dst_v)
    plsc.subcore_barrier()

    def gather(j, b):
        pltpu.async_copy(nodes_hbm.at[src_v.at[j]], bufs[b], gsems[b])

    def wait_gather(j, b):
        pltpu.make_async_copy(nodes_hbm.at[src_v.at[j]],
                              bufs[b], gsems[b]).wait()

    def scatter(j, b):
        pltpu.async_copy(bufs[b], acc.at[dst_v.at[j]], ssems[b], add=True)

    def wait_scatter(j, b):
        pltpu.make_async_copy(bufs[b], acc.at[dst_v.at[j]], ssems[b]).wait()

    for b in range(3):
        gather(b, b)

    @pl.loop(0, CHUNKS_PER_W, step=NB)
    def _(g):
        for b in range(NB):
            j = g + b
            b2 = (b + 3) % NB
            wait_gather(j, b)
            scatter(j, b)

            @pl.when(j < 2)
            def _fill():
                gather(j + 3, b2)

            @pl.when((j >= 2) & (j + 3 < CHUNKS_PER_W))
            def _recycle():
                wait_scatter(j - 2, b2)
                gather(j + 3, b2)

            @pl.when((j >= 2) & (j + 3 >= CHUNKS_PER_W))
            def _drain():
                wait_scatter(j - 2, b2)

    wait_scatter(CHUNKS_PER_W - 2, (CHUNKS_PER_W - 2) % NB)
    wait_scatter(CHUNKS_PER_W - 1, (CHUNKS_PER_W - 1) % NB)

    plsc.subcore_barrier()
    pltpu.sync_copy(acc.at[pl.ds(row0, ROWS_PER_TILE)],
                    out_hbm.at[c, pl.ds(row0, ROWS_PER_TILE)])


def _combine_body(p_ref, o_ref):
    o_ref[...] = p_ref[0] + p_ref[1]


_ROWS_BLK = 1000


def _combine(partials):
    return pl.pallas_call(
        _combine_body,
        grid=(N_NODES // _ROWS_BLK,),
        in_specs=[pl.BlockSpec((NC, _ROWS_BLK, D_FEAT), lambda i: (0, i, 0))],
        out_specs=pl.BlockSpec((_ROWS_BLK, D_FEAT), lambda i: (i, 0)),
        out_shape=jax.ShapeDtypeStruct((N_NODES, D_FEAT), jnp.float32),
    )(partials)


def kernel(node_features, edge_index):
    edges = edge_index.astype(jnp.int32).reshape(2, N_EDGES // CHUNK, CHUNK)
    partials = _sc_aggregate(node_features, edges)
    return _combine(partials)
